# Initial kernel scaffold; baseline (speedup 1.0000x reference)
import functools

import jax
import jax.numpy as jnp
from jax import lax
from jax.experimental import pallas as pl
from jax.experimental.pallas import tpu as pltpu
from jax.experimental.pallas import tpu_sc as plsc

N = 10000
E = 320000
D_IN = 128
HID = 64
HEADS = 8
NCLS = 128

NB = 10
BN = N // NB
NSC = 2
NT = 16
EPT = E // (NSC * NT)
CH = 128
NFULL = EPT // CH
TAIL = EPT - NFULL * CH
RW = 128
ZR = 125

_f32 = jnp.float32
_i32 = jnp.int32


def _sc_mesh():
    return plsc.VectorSubcoreMesh(
        core_axis_name="c", subcore_axis_name="s",
        num_cores=NSC, num_subcores=NT)


def _sc_params():
    import dataclasses
    cp = pltpu.CompilerParams()
    if "needs_layout_passes" in pltpu.CompilerParams.__dataclass_fields__:
        cp = dataclasses.replace(cp, needs_layout_passes=False)
    return cp


def _leaky(v):
    return jnp.maximum(v, 0.2 * v)


def _k1_body(x_ref, w1_ref, as_ref, ad_ref, h0, h1, h2, h3, asad_ref):
    h = jnp.dot(x_ref[...], w1_ref[...], preferred_element_type=_f32)
    for p, hr in enumerate((h0, h1, h2, h3)):
        hr[...] = h[:, p * RW:(p + 1) * RW]
    h3d = h.reshape(BN, HEADS, HID)
    asad_ref[0] = jnp.sum(h3d * as_ref[...][None], axis=-1).T
    asad_ref[1] = jnp.sum(h3d * ad_ref[...][None], axis=-1).T


_k1 = pl.pallas_call(
    _k1_body,
    grid=(NB,),
    in_specs=[
        pl.BlockSpec((BN, D_IN), lambda i: (i, 0)),
        pl.BlockSpec((D_IN, HEADS * HID), lambda i: (0, 0)),
        pl.BlockSpec((HEADS, HID), lambda i: (0, 0)),
        pl.BlockSpec((HEADS, HID), lambda i: (0, 0)),
    ],
    out_specs=[pl.BlockSpec((BN, RW), lambda i: (i, 0))] * 4
    + [pl.BlockSpec((2, HEADS, BN), lambda i: (0, 0, i))],
    out_shape=[jax.ShapeDtypeStruct((N, RW), _f32)] * 4
    + [jax.ShapeDtypeStruct((2, HEADS, N), _f32)],
)


def _make_stats(nh):
    flat = nh * N
    if (flat // NT) % 8 == 0:
        ntz, span = NT, flat // NT
    else:
        ntz, span = NT // 2, flat // (NT // 2)
    assert ntz * span == flat and span % 8 == 0

    scratch = [
        pltpu.VMEM((EPT,), _i32),
        pltpu.VMEM((EPT,), _i32),
        pltpu.VMEM((N,), _f32),
        pltpu.VMEM((N,), _f32),
        pltpu.VMEM((EPT,), _f32),
        pltpu.VMEM((NFULL + 1, CH), _f32),
        pltpu.VMEM((NFULL + 1, CH), _i32),
        pltpu.VMEM((span,), _f32),
        pltpu.VMEM_SHARED((flat,), _f32),
    ]

    @functools.partial(
        pl.kernel,
        out_type=[jax.ShapeDtypeStruct((nh * E,), _f32),
                  jax.ShapeDtypeStruct((NSC * flat,), _f32)],
        mesh=_sc_mesh(),
        scratch_types=scratch,
        compiler_params=_sc_params(),
    )
    def stats(src_hbm, dst_hbm, asad_hbm, ex_hbm, dp_hbm,
              src_full, dst_full, as_v, ad_v, ex_full, ex2d, adj2d, zb,
              den_sh):
        c = lax.axis_index("c")
        s = lax.axis_index("s")
        base0 = (c * NT + s) * EPT

        @pl.loop(0, span, step=16)
        def _(i):
            zb[pl.ds(i, 16)] = jnp.zeros((16,), _f32)

        @pl.when(s < ntz)
        def _():
            pltpu.sync_copy(zb, den_sh.at[pl.ds(s * span, span)])

        pltpu.sync_copy(src_hbm.at[pl.ds(base0, EPT)], src_full)
        pltpu.sync_copy(dst_hbm.at[pl.ds(base0, EPT)], dst_full)

        @pl.loop(TAIL, CH, step=16)
        def _(i):
            ex2d[NFULL, pl.ds(i, 16)] = jnp.zeros((16,), _f32)
            adj2d[NFULL, pl.ds(i, 16)] = jnp.zeros((16,), _i32)

        plsc.subcore_barrier()

        for h in range(nh):
            pltpu.sync_copy(asad_hbm.at[pl.ds(h * N, N)], as_v)
            pltpu.sync_copy(asad_hbm.at[pl.ds((nh + h) * N, N)], ad_v)

            def compute(ci, n, h=h):
                @pl.loop(0, n, step=16)
                def _(i):
                    g = ci * CH + i
                    sv = src_full[pl.ds(g, 16)]
                    dv = dst_full[pl.ds(g, 16)]
                    av = plsc.load_gather(as_v, [sv])
                    bv = plsc.load_gather(ad_v, [dv])
                    ex = jnp.exp(_leaky(av + bv))
                    ex_full[pl.ds(g, 16)] = ex
                    ex2d[ci, pl.ds(i, 16)] = ex
                    adj2d[ci, pl.ds(i, 16)] = dv + h * N

            @pl.loop(0, NFULL)
            def _(ci):
                compute(ci, CH)

            compute(NFULL, TAIL)

            pltpu.sync_copy(ex_full, ex_hbm.at[pl.ds(h * E + base0, EPT)])
            pltpu.sync_copy(ex2d, den_sh.at[adj2d], add=True)

        plsc.subcore_barrier()

        @pl.when(s < ntz)
        def _():
            pltpu.sync_copy(den_sh.at[pl.ds(s * span, span)],
                            dp_hbm.at[pl.ds(c * flat + s * span, span)])

    return stats


_stats8 = _make_stats(HEADS)
_stats1 = _make_stats(1)


def _make_heavy(npass, dual):
    scratch = [
        pltpu.VMEM((EPT,), _i32),
        pltpu.VMEM((EPT,), _i32),
        pltpu.VMEM((EPT,), _f32),
        pltpu.VMEM((EPT,), _f32),
        pltpu.VMEM((CH,), _i32),
        pltpu.VMEM((CH,), _i32),
        pltpu.VMEM((TAIL,), _i32),
        pltpu.VMEM((CH, RW), _f32),
        pltpu.VMEM((CH, RW), _f32),
        pltpu.VMEM((ZR, RW), _f32),
        pltpu.SemaphoreType.DMA,
        pltpu.SemaphoreType.DMA,
        pltpu.VMEM_SHARED((N, RW), _f32),
    ]

    @functools.partial(
        pl.kernel,
        out_type=[jax.ShapeDtypeStruct((NSC * N, RW), _f32)
                  for _ in range(npass)],
        mesh=_sc_mesh(),
        scratch_types=scratch,
        compiler_params=_sc_params(),
    )
    def heavy(src_hbm, dst_hbm, ex_hbm, *rest):
        h_hbms = rest[:npass]
        accs = rest[npass:2 * npass]
        (src_full, dst_full, exa_full, exb_full, dst_i0, dst_i1, dst_t,
         rows0, rows1, zrows, gsem0, gsem1, acc_sh) = rest[2 * npass:]
        c = lax.axis_index("c")
        s = lax.axis_index("s")
        base0 = (c * NT + s) * EPT
        rpt = N // NT

        @pl.loop(0, ZR)
        def _(r):
            for k in range(RW // 16):
                zrows[r, pl.ds(16 * k, 16)] = jnp.zeros((16,), _f32)

        pltpu.sync_copy(src_hbm.at[pl.ds(base0, EPT)], src_full)
        pltpu.sync_copy(dst_hbm.at[pl.ds(base0, EPT)], dst_full)

        for p in range(npass):
            for k in range(rpt // ZR):
                pltpu.sync_copy(
                    zrows, acc_sh.at[pl.ds(s * rpt + k * ZR, ZR)])

            pltpu.sync_copy(ex_hbm.at[pl.ds((2 * p if dual else 0) * E
                                            + base0, EPT)], exa_full)
            if dual:
                pltpu.sync_copy(ex_hbm.at[pl.ds((2 * p + 1) * E + base0,
                                                EPT)], exb_full)
            plsc.subcore_barrier()

            def vcopy_dst(ci, dref):
                for k in range(CH // 16):
                    dref[pl.ds(16 * k, 16)] = \
                        dst_full[pl.ds(ci * CH + 16 * k, 16)]

            def gather(ci, rows, sem, p=p):
                return pltpu.async_copy(
                    h_hbms[p].at[src_full.at[pl.ds(ci * CH, CH)]], rows, sem)

            def gwait(ci, rows, sem, p=p):
                pltpu.make_async_copy(
                    h_hbms[p].at[src_full.at[pl.ds(ci * CH, CH)]],
                    rows, sem).wait()

            def scale(ci, rows, n):
                @pl.loop(0, n)
                def _(e):
                    idx = jnp.full((16,), ci * CH + e, _i32)
                    sa = plsc.load_gather(exa_full, [idx])
                    sb = plsc.load_gather(exb_full, [idx]) if dual else sa
                    for k in range(RW // 16):
                        sk = sa if (not dual or k < RW // 32) else sb
                        rows[e, pl.ds(16 * k, 16)] = \
                            rows[e, pl.ds(16 * k, 16)] * sk

            vcopy_dst(0, dst_i0)
            gather(0, rows0, gsem0)

            @pl.loop(0, NFULL // 2)
            def _(i):
                b0 = 2 * i
                b1 = 2 * i + 1
                vcopy_dst(b1, dst_i1)
                gather(b1, rows1, gsem1)
                gwait(b0, rows0, gsem0)
                scale(b0, rows0, CH)
                pltpu.sync_copy(rows0, acc_sh.at[dst_i0], add=True)

                @pl.when(i < NFULL // 2 - 1)
                def _():
                    vcopy_dst(b0 + 2, dst_i0)
                    gather(b0 + 2, rows0, gsem0)

                gwait(b1, rows1, gsem1)
                scale(b1, rows1, CH)
                pltpu.sync_copy(rows1, acc_sh.at[dst_i1], add=True)

            for k in range(TAIL // 16):
                dst_t[pl.ds(16 * k, 16)] = \
                    dst_full[pl.ds(NFULL * CH + 16 * k, 16)]
            pltpu.sync_copy(
                h_hbms[p].at[src_full.at[pl.ds(NFULL * CH, TAIL)]],
                rows0.at[pl.ds(0, TAIL)])
            scale(NFULL, rows0, TAIL)
            pltpu.sync_copy(rows0.at[pl.ds(0, TAIL)], acc_sh.at[dst_t],
                            add=True)

            plsc.subcore_barrier()
            pltpu.sync_copy(acc_sh.at[pl.ds(s * rpt, rpt)],
                            accs[p].at[pl.ds(c * N + s * rpt, rpt)])

    return heavy


_heavy4 = _make_heavy(4, True)
_heavy1 = _make_heavy(1, False)


def _k4_body(a0, a1, a2, a3, h0, h1, h2, h3, asad_ref, dp_ref, w2_ref,
             as2_ref, ad2_ref, b1_ref, z2_ref, asad2_ref):
    exl = jnp.exp(_leaky(asad_ref[0] + asad_ref[1]))
    den = dp_ref[0] + dp_ref[1] + exl + 1e-16
    cols = []
    for p, (ar, hr) in enumerate(zip((a0, a1, a2, a3), (h0, h1, h2, h3))):
        num = ar[0] + ar[1]
        ht = hr[...]
        for j in range(2):
            h = 2 * p + j
            sl = slice(HID * j, HID * (j + 1))
            numh = num[:, sl] + exl[h][:, None] * ht[:, sl]
            cols.append(numh / den[h][:, None])
    hcat = jnp.concatenate(cols, axis=1) + b1_ref[...]
    hcat = jnp.where(hcat > 0, hcat,
                     jnp.exp(jnp.minimum(hcat, 0.)) - 1.)
    z2 = jnp.dot(hcat, w2_ref[...], preferred_element_type=_f32)
    z2_ref[...] = z2
    asad2_ref[0] = jnp.sum(z2 * as2_ref[...], axis=1)
    asad2_ref[1] = jnp.sum(z2 * ad2_ref[...], axis=1)


_k4 = pl.pallas_call(
    _k4_body,
    grid=(NB,),
    in_specs=[pl.BlockSpec((NSC, BN, RW), lambda i: (0, i, 0))] * 4
    + [pl.BlockSpec((BN, RW), lambda i: (i, 0))] * 4
    + [
        pl.BlockSpec((2, HEADS, BN), lambda i: (0, 0, i)),
        pl.BlockSpec((NSC, HEADS, BN), lambda i: (0, 0, i)),
        pl.BlockSpec((HEADS * HID, NCLS), lambda i: (0, 0)),
        pl.BlockSpec((1, NCLS), lambda i: (0, 0)),
        pl.BlockSpec((1, NCLS), lambda i: (0, 0)),
        pl.BlockSpec((1, HEADS * HID), lambda i: (0, 0)),
    ],
    out_specs=[
        pl.BlockSpec((BN, NCLS), lambda i: (i, 0)),
        pl.BlockSpec((2, BN), lambda i: (0, i)),
    ],
    out_shape=[
        jax.ShapeDtypeStruct((N, NCLS), _f32),
        jax.ShapeDtypeStruct((2, N), _f32),
    ],
)


def _k7_body(acc_ref, dp_ref, asad2_ref, z2_ref, b2_ref, out_ref):
    exl = jnp.exp(_leaky(asad2_ref[0] + asad2_ref[1]))
    den = dp_ref[0] + dp_ref[1] + exl + 1e-16
    num = acc_ref[0] + acc_ref[1] + exl[:, None] * z2_ref[...]
    out_ref[...] = num / den[:, None] + b2_ref[...]


_k7 = pl.pallas_call(
    _k7_body,
    grid=(NB,),
    in_specs=[
        pl.BlockSpec((NSC, BN, NCLS), lambda i: (0, i, 0)),
        pl.BlockSpec((NSC, BN), lambda i: (0, i)),
        pl.BlockSpec((2, BN), lambda i: (0, i)),
        pl.BlockSpec((BN, NCLS), lambda i: (i, 0)),
        pl.BlockSpec((1, NCLS), lambda i: (0, 0)),
    ],
    out_specs=pl.BlockSpec((BN, NCLS), lambda i: (i, 0)),
    out_shape=jax.ShapeDtypeStruct((N, NCLS), _f32),
)


def kernel(x, edge_index, W1, a_src1, a_dst1, b1, W2, a_src2, a_dst2, b2):
    src = edge_index[0]
    dst = edge_index[1]

    h0, h1, h2, h3, asad1 = _k1(x, W1, a_src1, a_dst1)
    ex1, dp1 = _stats8(src, dst, asad1.reshape(-1))
    accs = _heavy4(src, dst, ex1, h0, h1, h2, h3)
    z2, asad2 = _k4(*(a.reshape(NSC, N, RW) for a in accs),
                    h0, h1, h2, h3, asad1,
                    dp1.reshape(NSC, HEADS, N), W2, a_src2, a_dst2,
                    b1.reshape(1, -1))
    ex2, dp2 = _stats1(src, dst, asad2.reshape(-1))
    (acc2,) = _heavy1(src, dst, ex2, z2)
    out = _k7(acc2.reshape(NSC, N, NCLS), dp2.reshape(NSC, N), asad2, z2,
              b2.reshape(1, -1))
    return out


# speedup vs baseline: 41.8672x; 41.8672x over previous
import functools

import jax
import jax.numpy as jnp
from jax import lax
from jax.experimental import pallas as pl
from jax.experimental.pallas import tpu as pltpu
from jax.experimental.pallas import tpu_sc as plsc

N = 10000
E = 320000
D_IN = 128
HID = 64
HEADS = 8
NCLS = 128

NB = 10
BN = N // NB
NSC = 2
NT = 16
EPT = E // (NSC * NT)
CH = 128
NFULL = EPT // CH
TAIL = EPT - NFULL * CH
RW = 128

NTD = 10
RPD = N // NTD
PIECES = [(k * 128, 128) for k in range(7)] + [(896, 104)]

_f32 = jnp.float32
_i32 = jnp.int32


def _sc_mesh():
    return plsc.VectorSubcoreMesh(
        core_axis_name="c", subcore_axis_name="s",
        num_cores=NSC, num_subcores=NT)


def _sc_params():
    import dataclasses
    cp = pltpu.CompilerParams()
    if "needs_layout_passes" in pltpu.CompilerParams.__dataclass_fields__:
        cp = dataclasses.replace(cp, needs_layout_passes=False)
    return cp


def _leaky(v):
    return jnp.maximum(v, 0.2 * v)


def _k1_body(x_ref, w1_ref, as_ref, ad_ref, h0, h1, h2, h3, asad_ref):
    h = jnp.dot(x_ref[...], w1_ref[...], preferred_element_type=_f32)
    for p, hr in enumerate((h0, h1, h2, h3)):
        hr[...] = h[:, p * RW:(p + 1) * RW]
    h3d = h.reshape(BN, HEADS, HID)
    asad_ref[0, 0] = jnp.sum(h3d * as_ref[...][None], axis=-1).T
    asad_ref[0, 1] = jnp.sum(h3d * ad_ref[...][None], axis=-1).T


_k1 = pl.pallas_call(
    _k1_body,
    grid=(NB,),
    in_specs=[
        pl.BlockSpec((BN, D_IN), lambda i: (i, 0)),
        pl.BlockSpec((D_IN, HEADS * HID), lambda i: (0, 0)),
        pl.BlockSpec((HEADS, HID), lambda i: (0, 0)),
        pl.BlockSpec((HEADS, HID), lambda i: (0, 0)),
    ],
    out_specs=[pl.BlockSpec((BN, RW), lambda i: (i, 0))] * 4
    + [pl.BlockSpec((1, 2, HEADS, BN), lambda i: (i, 0, 0, 0))],
    out_shape=[jax.ShapeDtypeStruct((N, RW), _f32)] * 4
    + [jax.ShapeDtypeStruct((NB, 2, HEADS, BN), _f32)],
)


def _make_stats(nh):
    flat = nh * N
    groups = nh * NB
    if groups >= NT:
        ntz, gpt = NT, groups // NT
    else:
        ntz, gpt = groups, 1
    assert ntz * gpt == groups
    spanz = flat // ntz
    assert spanz % 8 == 0 and spanz >= BN

    scratch = [
        pltpu.VMEM((EPT,), _i32),
        pltpu.VMEM((EPT,), _i32),
        pltpu.VMEM((EPT,), _i32),
        pltpu.VMEM((N,), _f32),
        pltpu.VMEM((N,), _f32),
        pltpu.VMEM((EPT,), _f32),
        pltpu.VMEM((NFULL + 1, CH), _f32),
        pltpu.VMEM((NFULL + 1, CH), _i32),
        pltpu.VMEM((spanz,), _f32),
        pltpu.SemaphoreType.DMA,
        pltpu.VMEM_SHARED((flat,), _f32),
    ]

    @functools.partial(
        pl.kernel,
        out_type=[jax.ShapeDtypeStruct((nh * E,), _f32),
                  jax.ShapeDtypeStruct((NSC * flat,), _f32)],
        mesh=_sc_mesh(),
        scratch_types=scratch,
        compiler_params=_sc_params(),
    )
    def stats(src_hbm, dst_hbm, asad_hbm, ex_hbm, dp_hbm,
              src_full, dst_full, badj_full, as_v, ad_v, ex_full, ex2d,
              adj2d, zb, ssem, den_sh):
        c = lax.axis_index("c")
        s = lax.axis_index("s")
        base0 = (c * NT + s) * EPT

        @pl.loop(0, spanz, step=16)
        def _(i):
            zb[pl.ds(i, 16)] = jnp.zeros((16,), _f32)

        @pl.when(s < ntz)
        def _():
            pltpu.sync_copy(zb, den_sh.at[pl.ds(s * spanz, spanz)])

        pltpu.sync_copy(src_hbm.at[pl.ds(base0, EPT)], src_full)
        pltpu.sync_copy(dst_hbm.at[pl.ds(base0, EPT)], dst_full)

        @pl.loop(0, EPT, step=16)
        def _(i):
            dv = dst_full[pl.ds(i, 16)]
            badj_full[pl.ds(i, 16)] = \
                (dv // BN) * (nh * BN) + lax.rem(dv, BN)

        @pl.loop(TAIL, CH, step=16)
        def _(i):
            ex2d[NFULL, pl.ds(i, 16)] = jnp.zeros((16,), _f32)
            adj2d[NFULL, pl.ds(i, 16)] = jnp.zeros((16,), _i32)

        plsc.subcore_barrier()

        for h in range(nh):
            for b in range(NB):
                pltpu.sync_copy(
                    asad_hbm.at[pl.ds(((b * 2 + 0) * nh + h) * BN, BN)],
                    as_v.at[pl.ds(b * BN, BN)])
                pltpu.sync_copy(
                    asad_hbm.at[pl.ds(((b * 2 + 1) * nh + h) * BN, BN)],
                    ad_v.at[pl.ds(b * BN, BN)])

            def compute(ci, n, h=h):
                @pl.loop(0, n, step=16)
                def _(i):
                    g = ci * CH + i
                    sv = src_full[pl.ds(g, 16)]
                    dv = dst_full[pl.ds(g, 16)]
                    av = plsc.load_gather(as_v, [sv])
                    bv = plsc.load_gather(ad_v, [dv])
                    ex = jnp.exp(_leaky(av + bv))
                    ex_full[pl.ds(g, 16)] = ex
                    ex2d[ci, pl.ds(i, 16)] = ex
                    adj2d[ci, pl.ds(i, 16)] = \
                        badj_full[pl.ds(g, 16)] + h * BN

            @pl.loop(0, NFULL)
            def _(ci):
                compute(ci, CH)

            compute(NFULL, TAIL)

            pltpu.sync_copy(ex_full, ex_hbm.at[pl.ds(h * E + base0, EPT)])

            @pl.loop(0, NFULL + 1)
            def _(ci):
                pltpu.async_copy(ex2d.at[ci], den_sh.at[adj2d.at[ci]],
                                 ssem, add=True)

            @pl.loop(0, NFULL + 1)
            def _(ci):
                pltpu.make_async_copy(ex2d.at[0], den_sh.at[adj2d.at[0]],
                                      ssem).wait()

        plsc.subcore_barrier()

        @pl.when(s < ntz)
        def _():
            for j in range(gpt):
                g = s * gpt + j
                pltpu.sync_copy(den_sh.at[pl.ds(g * BN, BN)],
                                zb.at[pl.ds(0, BN)])
                pltpu.sync_copy(zb.at[pl.ds(0, BN)],
                                dp_hbm.at[pl.ds((g * NSC + c) * BN, BN)])

    return stats


_stats8 = _make_stats(HEADS)
_stats1 = _make_stats(1)


def _make_heavy(npass, dual):
    scratch = [
        pltpu.VMEM((EPT,), _i32),
        pltpu.VMEM((CH,), _i32),
        pltpu.VMEM((CH,), _i32),
        pltpu.VMEM((CH,), _f32),
        pltpu.VMEM((CH,), _f32),
        pltpu.VMEM((CH,), _f32),
        pltpu.VMEM((CH,), _f32),
        pltpu.VMEM((TAIL,), _i32),
        pltpu.VMEM((TAIL,), _f32),
        pltpu.VMEM((TAIL,), _f32),
        pltpu.VMEM((CH, RW), _f32),
        pltpu.VMEM((CH, RW), _f32),
        pltpu.SemaphoreType.DMA,
        pltpu.SemaphoreType.DMA,
        pltpu.VMEM_SHARED((N, RW), _f32),
    ]

    def _zero_rows(rows):
        @pl.loop(0, CH)
        def _(r):
            for k in range(RW // 16):
                rows[r, pl.ds(16 * k, 16)] = jnp.zeros((16,), _f32)

    @functools.partial(
        pl.kernel,
        out_type=[jax.ShapeDtypeStruct((NSC * N, RW), _f32)
                  for _ in range(npass)],
        mesh=_sc_mesh(),
        scratch_types=scratch,
        compiler_params=_sc_params(),
    )
    def heavy(src_hbm, dst_hbm, ex_hbm, *rest):
        h_hbms = rest[:npass]
        accs = rest[npass:2 * npass]
        (src_full, dst_i0, dst_i1, exa_i0, exa_i1, exb_i0, exb_i1,
         dst_t, exa_t, exb_t, rows0, rows1, gsem0, gsem1,
         acc_sh) = rest[2 * npass:]
        c = lax.axis_index("c")
        s = lax.axis_index("s")
        base0 = (c * NT + s) * EPT

        pltpu.sync_copy(src_hbm.at[pl.ds(base0, EPT)], src_full)

        for p in range(npass):
            ha = 2 * p if dual else 0

            _zero_rows(rows0)

            @pl.when(s < NTD)
            def _():
                for off, sz in PIECES:
                    pltpu.sync_copy(
                        rows0.at[pl.ds(0, sz)],
                        acc_sh.at[pl.ds(s * RPD + off, sz)])

            plsc.subcore_barrier()

            def bparts(ci, rows, dref, ear, ebr, n, p=p, ha=ha):
                base = base0 + ci * CH
                parts = [
                    (h_hbms[p].at[src_full.at[pl.ds(ci * CH, n)]],
                     rows.at[pl.ds(0, n)] if n != CH else rows),
                    (dst_hbm.at[pl.ds(base, n)], dref),
                    (ex_hbm.at[pl.ds(ha * E + base, n)], ear),
                ]
                if dual:
                    parts.append(
                        (ex_hbm.at[pl.ds((ha + 1) * E + base, n)], ebr))
                return parts

            def issue(ci, rows, dref, ear, ebr, sem, n=CH):
                for sr, dr in bparts(ci, rows, dref, ear, ebr, n):
                    pltpu.async_copy(sr, dr, sem)

            def drain(ci, rows, dref, ear, ebr, sem, n=CH):
                for sr, dr in bparts(ci, rows, dref, ear, ebr, n):
                    pltpu.make_async_copy(sr, dr, sem).wait()

            def scale(rows, ear, ebr, n):
                @pl.loop(0, n)
                def _(e):
                    idx = jnp.full((16,), e, _i32)
                    sa = plsc.load_gather(ear, [idx])
                    sb = plsc.load_gather(ebr, [idx]) if dual else sa
                    for k in range(RW // 16):
                        sk = sa if (not dual or k < RW // 32) else sb
                        rows[e, pl.ds(16 * k, 16)] = \
                            rows[e, pl.ds(16 * k, 16)] * sk

            set0 = (rows0, dst_i0, exa_i0, exb_i0)
            set1 = (rows1, dst_i1, exa_i1, exb_i1)

            issue(0, *set0, gsem0)

            @pl.loop(0, NFULL // 2)
            def _(i):
                b0 = 2 * i
                b1 = 2 * i + 1
                issue(b1, *set1, gsem1)
                drain(b0, *set0, gsem0)
                scale(rows0, exa_i0, exb_i0, CH)
                pltpu.sync_copy(rows0, acc_sh.at[dst_i0], add=True)

                @pl.when(i < NFULL // 2 - 1)
                def _():
                    issue(b0 + 2, *set0, gsem0)

                drain(b1, *set1, gsem1)
                scale(rows1, exa_i1, exb_i1, CH)
                pltpu.sync_copy(rows1, acc_sh.at[dst_i1], add=True)

            issue(NFULL, rows0, dst_t, exa_t, exb_t, gsem0, TAIL)
            drain(NFULL, rows0, dst_t, exa_t, exb_t, gsem0, TAIL)
            scale(rows0, exa_t, exb_t, TAIL)
            pltpu.sync_copy(rows0.at[pl.ds(0, TAIL)], acc_sh.at[dst_t],
                            add=True)

            plsc.subcore_barrier()

            @pl.when(s < NTD)
            def _():
                for off, sz in PIECES:
                    pltpu.sync_copy(acc_sh.at[pl.ds(s * RPD + off, sz)],
                                    rows1.at[pl.ds(0, sz)])
                    pltpu.sync_copy(
                        rows1.at[pl.ds(0, sz)],
                        accs[p].at[pl.ds(c * N + s * RPD + off, sz)])

    return heavy


_heavy4 = _make_heavy(4, True)
_heavy1 = _make_heavy(1, False)


def _k4_body(a0, a1, a2, a3, h0, h1, h2, h3, asad_ref, dp_ref, w2_ref,
             as2_ref, ad2_ref, b1_ref, z2_ref, asad2_ref):
    exl = jnp.exp(_leaky(asad_ref[0, 0] + asad_ref[0, 1]))
    den = dp_ref[0, :, 0, :] + dp_ref[0, :, 1, :] + exl + 1e-16
    cols = []
    for p, (ar, hr) in enumerate(zip((a0, a1, a2, a3), (h0, h1, h2, h3))):
        num = ar[0] + ar[1]
        ht = hr[...]
        for j in range(2):
            h = 2 * p + j
            sl = slice(HID * j, HID * (j + 1))
            numh = num[:, sl] + exl[h][:, None] * ht[:, sl]
            cols.append(numh / den[h][:, None])
    hcat = jnp.concatenate(cols, axis=1) + b1_ref[...]
    hcat = jnp.where(hcat > 0, hcat,
                     jnp.exp(jnp.minimum(hcat, 0.)) - 1.)
    z2 = jnp.dot(hcat, w2_ref[...], preferred_element_type=_f32)
    z2_ref[...] = z2
    asad2_ref[0, 0] = jnp.sum(z2 * as2_ref[...], axis=1)
    asad2_ref[0, 1] = jnp.sum(z2 * ad2_ref[...], axis=1)


_k4 = pl.pallas_call(
    _k4_body,
    grid=(NB,),
    in_specs=[pl.BlockSpec((NSC, BN, RW), lambda i: (0, i, 0))] * 4
    + [pl.BlockSpec((BN, RW), lambda i: (i, 0))] * 4
    + [
        pl.BlockSpec((1, 2, HEADS, BN), lambda i: (i, 0, 0, 0)),
        pl.BlockSpec((1, HEADS, NSC, BN), lambda i: (i, 0, 0, 0)),
        pl.BlockSpec((HEADS * HID, NCLS), lambda i: (0, 0)),
        pl.BlockSpec((1, NCLS), lambda i: (0, 0)),
        pl.BlockSpec((1, NCLS), lambda i: (0, 0)),
        pl.BlockSpec((1, HEADS * HID), lambda i: (0, 0)),
    ],
    out_specs=[
        pl.BlockSpec((BN, NCLS), lambda i: (i, 0)),
        pl.BlockSpec((1, 2, BN), lambda i: (i, 0, 0)),
    ],
    out_shape=[
        jax.ShapeDtypeStruct((N, NCLS), _f32),
        jax.ShapeDtypeStruct((NB, 2, BN), _f32),
    ],
)


def _k7_body(acc_ref, dp_ref, asad2_ref, z2_ref, b2_ref, out_ref):
    exl = jnp.exp(_leaky(asad2_ref[0, 0] + asad2_ref[0, 1]))
    den = dp_ref[0, 0] + dp_ref[0, 1] + exl + 1e-16
    num = acc_ref[0] + acc_ref[1] + exl[:, None] * z2_ref[...]
    out_ref[...] = num / den[:, None] + b2_ref[...]


_k7 = pl.pallas_call(
    _k7_body,
    grid=(NB,),
    in_specs=[
        pl.BlockSpec((NSC, BN, NCLS), lambda i: (0, i, 0)),
        pl.BlockSpec((1, NSC, BN), lambda i: (i, 0, 0)),
        pl.BlockSpec((1, 2, BN), lambda i: (i, 0, 0)),
        pl.BlockSpec((BN, NCLS), lambda i: (i, 0)),
        pl.BlockSpec((1, NCLS), lambda i: (0, 0)),
    ],
    out_specs=pl.BlockSpec((BN, NCLS), lambda i: (i, 0)),
    out_shape=jax.ShapeDtypeStruct((N, NCLS), _f32),
)


def kernel(x, edge_index, W1, a_src1, a_dst1, b1, W2, a_src2, a_dst2, b2):
    src = edge_index[0]
    dst = edge_index[1]

    h0, h1, h2, h3, asad1 = _k1(x, W1, a_src1, a_dst1)
    ex1, dp1 = _stats8(src, dst, asad1.reshape(-1))
    accs = _heavy4(src, dst, ex1, h0, h1, h2, h3)
    z2, asad2 = _k4(*(a.reshape(NSC, N, RW) for a in accs),
                    h0, h1, h2, h3, asad1,
                    dp1.reshape(NB, HEADS, NSC, BN), W2, a_src2, a_dst2,
                    b1.reshape(1, -1))
    ex2, dp2 = _stats1(src, dst, asad2.reshape(-1))
    (acc2,) = _heavy1(src, dst, ex2, z2)
    out = _k7(acc2.reshape(NSC, N, NCLS), dp2.reshape(NB, NSC, BN), asad2,
              z2, b2.reshape(1, -1))
    return out


# speedup vs baseline: 54.5648x; 1.3033x over previous
import functools

import jax
import jax.numpy as jnp
from jax import lax
from jax.experimental import pallas as pl
from jax.experimental.pallas import tpu as pltpu
from jax.experimental.pallas import tpu_sc as plsc

N = 10000
E = 320000
D_IN = 128
HID = 64
HEADS = 8
NCLS = 128

NB = 10
BN = N // NB
NSC = 2
NT = 16
EPT = E // (NSC * NT)
CH = 128
NFULL = EPT // CH
TAIL = EPT - NFULL * CH
RW = 128

NTD = 10
RPD = N // NTD
PIECES = [(k * 128, 128) for k in range(7)] + [(896, 104)]

_f32 = jnp.float32
_i32 = jnp.int32


def _sc_mesh():
    return plsc.VectorSubcoreMesh(
        core_axis_name="c", subcore_axis_name="s",
        num_cores=NSC, num_subcores=NT)


def _sc_params():
    import dataclasses
    cp = pltpu.CompilerParams()
    if "needs_layout_passes" in pltpu.CompilerParams.__dataclass_fields__:
        cp = dataclasses.replace(cp, needs_layout_passes=False)
    return cp


def _leaky(v):
    return jnp.maximum(v, 0.2 * v)


def _lane_bcast(v, j):
    idx = jnp.full((16, 1), j, _i32)
    dnums = lax.GatherDimensionNumbers(
        offset_dims=(), collapsed_slice_dims=(0,), start_index_map=(0,))
    return lax.gather(v, idx, dnums, (1,),
                      mode=lax.GatherScatterMode.PROMISE_IN_BOUNDS)


def _k1_body(x_ref, w1_ref, as_ref, ad_ref, h0, h1, h2, h3, asad_ref):
    h = jnp.dot(x_ref[...], w1_ref[...], preferred_element_type=_f32)
    for p, hr in enumerate((h0, h1, h2, h3)):
        hr[...] = h[:, p * RW:(p + 1) * RW]
    h3d = h.reshape(BN, HEADS, HID)
    asad_ref[0, 0] = jnp.sum(h3d * as_ref[...][None], axis=-1).T
    asad_ref[0, 1] = jnp.sum(h3d * ad_ref[...][None], axis=-1).T


_k1 = pl.pallas_call(
    _k1_body,
    grid=(NB,),
    in_specs=[
        pl.BlockSpec((BN, D_IN), lambda i: (i, 0)),
        pl.BlockSpec((D_IN, HEADS * HID), lambda i: (0, 0)),
        pl.BlockSpec((HEADS, HID), lambda i: (0, 0)),
        pl.BlockSpec((HEADS, HID), lambda i: (0, 0)),
    ],
    out_specs=[pl.BlockSpec((BN, RW), lambda i: (i, 0))] * 4
    + [pl.BlockSpec((1, 2, HEADS, BN), lambda i: (i, 0, 0, 0))],
    out_shape=[jax.ShapeDtypeStruct((N, RW), _f32)] * 4
    + [jax.ShapeDtypeStruct((NB, 2, HEADS, BN), _f32)],
)


def _make_stats(nh):
    flat = nh * N
    groups = nh * NB
    if groups >= NT:
        ntz, gpt = NT, groups // NT
    else:
        ntz, gpt = groups, 1
    assert ntz * gpt == groups
    spanz = flat // ntz
    assert spanz % 8 == 0 and spanz >= BN

    scratch = [
        pltpu.VMEM((EPT,), _i32),
        pltpu.VMEM((EPT,), _i32),
        pltpu.VMEM((EPT,), _i32),
        pltpu.VMEM((N,), _f32),
        pltpu.VMEM((N,), _f32),
        pltpu.VMEM((EPT,), _f32),
        pltpu.VMEM((EPT,), _f32),
        pltpu.VMEM((NFULL + 1, CH), _f32),
        pltpu.VMEM((NFULL + 1, CH), _f32),
        pltpu.VMEM((NFULL + 1, CH), _i32),
        pltpu.VMEM((NFULL + 1, CH), _i32),
        pltpu.VMEM((spanz,), _f32),
        pltpu.SemaphoreType.DMA,
        pltpu.SemaphoreType.DMA,
        pltpu.SemaphoreType.DMA,
        pltpu.VMEM_SHARED((flat,), _f32),
    ]

    @functools.partial(
        pl.kernel,
        out_type=[jax.ShapeDtypeStruct((nh * E,), _f32),
                  jax.ShapeDtypeStruct((NSC * flat,), _f32)],
        mesh=_sc_mesh(),
        scratch_types=scratch,
        compiler_params=_sc_params(),
    )
    def stats(src_hbm, dst_hbm, asad_hbm, ex_hbm, dp_hbm,
              src_full, dst_full, badj_full, as_v, ad_v, ex_full_a,
              ex_full_b, ex2d_a, ex2d_b, adj2d_a, adj2d_b, zb,
              ssem, xsem, lsem, den_sh):
        exfs = (ex_full_a, ex_full_b)
        ex2ds = (ex2d_a, ex2d_b)
        adj2ds = (adj2d_a, adj2d_b)
        c = lax.axis_index("c")
        s = lax.axis_index("s")
        base0 = (c * NT + s) * EPT

        @pl.loop(0, spanz, step=16)
        def _(i):
            zb[pl.ds(i, 16)] = jnp.zeros((16,), _f32)

        @pl.when(s < ntz)
        def _():
            pltpu.sync_copy(zb, den_sh.at[pl.ds(s * spanz, spanz)])

        pltpu.sync_copy(src_hbm.at[pl.ds(base0, EPT)], src_full)
        pltpu.sync_copy(dst_hbm.at[pl.ds(base0, EPT)], dst_full)

        @pl.loop(0, EPT, step=16)
        def _(i):
            dv = dst_full[pl.ds(i, 16)]
            badj_full[pl.ds(i, 16)] = \
                (dv // BN) * (nh * BN) + lax.rem(dv, BN)

        @pl.loop(TAIL, CH, step=16)
        def _(i):
            for q in range(2):
                ex2ds[q][NFULL, pl.ds(i, 16)] = jnp.zeros((16,), _f32)
                adj2ds[q][NFULL, pl.ds(i, 16)] = jnp.zeros((16,), _i32)

        plsc.subcore_barrier()

        def load_parts(h):
            return [(asad_hbm.at[pl.ds(((b * 2 + q) * nh + h) * BN, BN)],
                     (as_v if q == 0 else ad_v).at[pl.ds(b * BN, BN)])
                    for b in range(NB) for q in range(2)]

        def drain_head(hq):
            @pl.loop(0, NFULL + 1)
            def _(ci):
                pltpu.make_async_copy(
                    ex2ds[hq].at[0], den_sh.at[adj2ds[hq].at[0]],
                    ssem).wait()
            pltpu.make_async_copy(
                exfs[hq], ex_hbm.at[pl.ds(base0, EPT)], xsem).wait()

        for h in range(nh):
            q = h % 2
            for sr, dr in load_parts(h):
                pltpu.async_copy(sr, dr, lsem)
            if h > 0:
                drain_head((h - 1) % 2)
            for sr, dr in load_parts(h):
                pltpu.make_async_copy(sr, dr, lsem).wait()

            exf, e2d, a2d = exfs[q], ex2ds[q], adj2ds[q]

            def compute(ci, n, h=h, exf=exf, e2d=e2d, a2d=a2d):
                @pl.loop(0, n, step=16)
                def _(i):
                    g = ci * CH + i
                    sv = src_full[pl.ds(g, 16)]
                    dv = dst_full[pl.ds(g, 16)]
                    av = plsc.load_gather(as_v, [sv])
                    bv = plsc.load_gather(ad_v, [dv])
                    ex = jnp.exp(_leaky(av + bv))
                    exf[pl.ds(g, 16)] = ex
                    e2d[ci, pl.ds(i, 16)] = ex
                    a2d[ci, pl.ds(i, 16)] = \
                        badj_full[pl.ds(g, 16)] + h * BN

            @pl.loop(0, NFULL)
            def _(ci):
                compute(ci, CH)

            compute(NFULL, TAIL)

            pltpu.async_copy(exf, ex_hbm.at[pl.ds(h * E + base0, EPT)],
                             xsem)

            @pl.loop(0, NFULL + 1)
            def _(ci):
                pltpu.async_copy(e2d.at[ci], den_sh.at[a2d.at[ci]],
                                 ssem, add=True)

        drain_head((nh - 1) % 2)
        plsc.subcore_barrier()

        @pl.when(s < ntz)
        def _():
            for j in range(gpt):
                g = s * gpt + j
                pltpu.sync_copy(den_sh.at[pl.ds(g * BN, BN)],
                                zb.at[pl.ds(0, BN)])
                pltpu.sync_copy(zb.at[pl.ds(0, BN)],
                                dp_hbm.at[pl.ds((g * NSC + c) * BN, BN)])

    return stats


_stats8 = _make_stats(HEADS)
_stats1 = _make_stats(1)


def _make_heavy(npass, dual):
    scratch = [
        pltpu.VMEM((EPT,), _i32),
        pltpu.VMEM((CH,), _i32),
        pltpu.VMEM((CH,), _i32),
        pltpu.VMEM((CH,), _f32),
        pltpu.VMEM((CH,), _f32),
        pltpu.VMEM((CH,), _f32),
        pltpu.VMEM((CH,), _f32),
        pltpu.VMEM((TAIL,), _i32),
        pltpu.VMEM((TAIL,), _f32),
        pltpu.VMEM((TAIL,), _f32),
        pltpu.VMEM((CH, RW), _f32),
        pltpu.VMEM((CH, RW), _f32),
        pltpu.SemaphoreType.DMA,
        pltpu.SemaphoreType.DMA,
        pltpu.VMEM_SHARED((N, RW), _f32),
    ]

    def _zero_rows(rows):
        @pl.loop(0, CH)
        def _(r):
            for k in range(RW // 16):
                rows[r, pl.ds(16 * k, 16)] = jnp.zeros((16,), _f32)

    @functools.partial(
        pl.kernel,
        out_type=[jax.ShapeDtypeStruct((NSC * N, RW), _f32)
                  for _ in range(npass)],
        mesh=_sc_mesh(),
        scratch_types=scratch,
        compiler_params=_sc_params(),
    )
    def heavy(src_hbm, dst_hbm, ex_hbm, *rest):
        h_hbms = rest[:npass]
        accs = rest[npass:2 * npass]
        (src_full, dst_i0, dst_i1, exa_i0, exa_i1, exb_i0, exb_i1,
         dst_t, exa_t, exb_t, rows0, rows1, gsem0, gsem1,
         acc_sh) = rest[2 * npass:]
        c = lax.axis_index("c")
        s = lax.axis_index("s")
        base0 = (c * NT + s) * EPT

        pltpu.sync_copy(src_hbm.at[pl.ds(base0, EPT)], src_full)

        for p in range(npass):
            ha = 2 * p if dual else 0

            _zero_rows(rows0)

            @pl.when(s < NTD)
            def _():
                for off, sz in PIECES:
                    pltpu.sync_copy(
                        rows0.at[pl.ds(0, sz)],
                        acc_sh.at[pl.ds(s * RPD + off, sz)])

            plsc.subcore_barrier()

            def bparts(ci, rows, dref, ear, ebr, n, p=p, ha=ha):
                base = base0 + ci * CH
                parts = [
                    (h_hbms[p].at[src_full.at[pl.ds(ci * CH, n)]],
                     rows.at[pl.ds(0, n)] if n != CH else rows),
                    (dst_hbm.at[pl.ds(base, n)], dref),
                    (ex_hbm.at[pl.ds(ha * E + base, n)], ear),
                ]
                if dual:
                    parts.append(
                        (ex_hbm.at[pl.ds((ha + 1) * E + base, n)], ebr))
                return parts

            def issue(ci, rows, dref, ear, ebr, sem, n=CH):
                for sr, dr in bparts(ci, rows, dref, ear, ebr, n):
                    pltpu.async_copy(sr, dr, sem)

            def drain(ci, rows, dref, ear, ebr, sem, n=CH):
                for sr, dr in bparts(ci, rows, dref, ear, ebr, n):
                    pltpu.make_async_copy(sr, dr, sem).wait()

            def scale(rows, ear, ebr, n):
                @pl.loop(0, n, step=16)
                def _(e0):
                    va = ear[pl.ds(e0, 16)]
                    vb = ebr[pl.ds(e0, 16)] if dual else va
                    for j in range(16):
                        sa = _lane_bcast(va, j)
                        sb = _lane_bcast(vb, j) if dual else sa
                        for k in range(RW // 16):
                            sk = sa if (not dual or k < RW // 32) else sb
                            rows[e0 + j, pl.ds(16 * k, 16)] = \
                                rows[e0 + j, pl.ds(16 * k, 16)] * sk

            set0 = (rows0, dst_i0, exa_i0, exb_i0)
            set1 = (rows1, dst_i1, exa_i1, exb_i1)

            issue(0, *set0, gsem0)

            @pl.loop(0, NFULL // 2)
            def _(i):
                b0 = 2 * i
                b1 = 2 * i + 1
                issue(b1, *set1, gsem1)
                drain(b0, *set0, gsem0)
                scale(rows0, exa_i0, exb_i0, CH)
                pltpu.sync_copy(rows0, acc_sh.at[dst_i0], add=True)

                @pl.when(i < NFULL // 2 - 1)
                def _():
                    issue(b0 + 2, *set0, gsem0)

                drain(b1, *set1, gsem1)
                scale(rows1, exa_i1, exb_i1, CH)
                pltpu.sync_copy(rows1, acc_sh.at[dst_i1], add=True)

            issue(NFULL, rows0, dst_t, exa_t, exb_t, gsem0, TAIL)
            drain(NFULL, rows0, dst_t, exa_t, exb_t, gsem0, TAIL)
            scale(rows0, exa_t, exb_t, TAIL)
            pltpu.sync_copy(rows0.at[pl.ds(0, TAIL)], acc_sh.at[dst_t],
                            add=True)

            plsc.subcore_barrier()

            @pl.when(s < NTD)
            def _():
                for off, sz in PIECES:
                    pltpu.sync_copy(acc_sh.at[pl.ds(s * RPD + off, sz)],
                                    rows1.at[pl.ds(0, sz)])
                    pltpu.sync_copy(
                        rows1.at[pl.ds(0, sz)],
                        accs[p].at[pl.ds(c * N + s * RPD + off, sz)])

    return heavy


_heavy4 = _make_heavy(4, True)
_heavy1 = _make_heavy(1, False)


def _k4_body(a0, a1, a2, a3, h0, h1, h2, h3, asad_ref, dp_ref, w2_ref,
             as2_ref, ad2_ref, b1_ref, z2_ref, asad2_ref):
    exl = jnp.exp(_leaky(asad_ref[0, 0] + asad_ref[0, 1]))
    den = dp_ref[0, :, 0, :] + dp_ref[0, :, 1, :] + exl + 1e-16
    cols = []
    for p, (ar, hr) in enumerate(zip((a0, a1, a2, a3), (h0, h1, h2, h3))):
        num = ar[0] + ar[1]
        ht = hr[...]
        for j in range(2):
            h = 2 * p + j
            sl = slice(HID * j, HID * (j + 1))
            numh = num[:, sl] + exl[h][:, None] * ht[:, sl]
            cols.append(numh / den[h][:, None])
    hcat = jnp.concatenate(cols, axis=1) + b1_ref[...]
    hcat = jnp.where(hcat > 0, hcat,
                     jnp.exp(jnp.minimum(hcat, 0.)) - 1.)
    z2 = jnp.dot(hcat, w2_ref[...], preferred_element_type=_f32)
    z2_ref[...] = z2
    asad2_ref[0, 0] = jnp.sum(z2 * as2_ref[...], axis=1)
    asad2_ref[0, 1] = jnp.sum(z2 * ad2_ref[...], axis=1)


_k4 = pl.pallas_call(
    _k4_body,
    grid=(NB,),
    in_specs=[pl.BlockSpec((NSC, BN, RW), lambda i: (0, i, 0))] * 4
    + [pl.BlockSpec((BN, RW), lambda i: (i, 0))] * 4
    + [
        pl.BlockSpec((1, 2, HEADS, BN), lambda i: (i, 0, 0, 0)),
        pl.BlockSpec((1, HEADS, NSC, BN), lambda i: (i, 0, 0, 0)),
        pl.BlockSpec((HEADS * HID, NCLS), lambda i: (0, 0)),
        pl.BlockSpec((1, NCLS), lambda i: (0, 0)),
        pl.BlockSpec((1, NCLS), lambda i: (0, 0)),
        pl.BlockSpec((1, HEADS * HID), lambda i: (0, 0)),
    ],
    out_specs=[
        pl.BlockSpec((BN, NCLS), lambda i: (i, 0)),
        pl.BlockSpec((1, 2, BN), lambda i: (i, 0, 0)),
    ],
    out_shape=[
        jax.ShapeDtypeStruct((N, NCLS), _f32),
        jax.ShapeDtypeStruct((NB, 2, BN), _f32),
    ],
)


def _k7_body(acc_ref, dp_ref, asad2_ref, z2_ref, b2_ref, out_ref):
    exl = jnp.exp(_leaky(asad2_ref[0, 0] + asad2_ref[0, 1]))
    den = dp_ref[0, 0] + dp_ref[0, 1] + exl + 1e-16
    num = acc_ref[0] + acc_ref[1] + exl[:, None] * z2_ref[...]
    out_ref[...] = num / den[:, None] + b2_ref[...]


_k7 = pl.pallas_call(
    _k7_body,
    grid=(NB,),
    in_specs=[
        pl.BlockSpec((NSC, BN, NCLS), lambda i: (0, i, 0)),
        pl.BlockSpec((1, NSC, BN), lambda i: (i, 0, 0)),
        pl.BlockSpec((1, 2, BN), lambda i: (i, 0, 0)),
        pl.BlockSpec((BN, NCLS), lambda i: (i, 0)),
        pl.BlockSpec((1, NCLS), lambda i: (0, 0)),
    ],
    out_specs=pl.BlockSpec((BN, NCLS), lambda i: (i, 0)),
    out_shape=jax.ShapeDtypeStruct((N, NCLS), _f32),
)


def kernel(x, edge_index, W1, a_src1, a_dst1, b1, W2, a_src2, a_dst2, b2):
    src = edge_index[0]
    dst = edge_index[1]

    h0, h1, h2, h3, asad1 = _k1(x, W1, a_src1, a_dst1)
    ex1, dp1 = _stats8(src, dst, asad1.reshape(-1))
    accs = _heavy4(src, dst, ex1, h0, h1, h2, h3)
    z2, asad2 = _k4(*(a.reshape(NSC, N, RW) for a in accs),
                    h0, h1, h2, h3, asad1,
                    dp1.reshape(NB, HEADS, NSC, BN), W2, a_src2, a_dst2,
                    b1.reshape(1, -1))
    ex2, dp2 = _stats1(src, dst, asad2.reshape(-1))
    (acc2,) = _heavy1(src, dst, ex2, z2)
    out = _k7(acc2.reshape(NSC, N, NCLS), dp2.reshape(NB, NSC, BN), asad2,
              z2, b2.reshape(1, -1))
    return out
